# baseline (speedup 1.0000x reference)
import functools
import math

import jax
import jax.numpy as jnp
from jax import lax
from jax.experimental import pallas as pl
from jax.experimental.pallas import tpu as pltpu
from jax.experimental.pallas import tpu_sc as plsc

SEQ, B, DIM = 4096, 8, 1024
LANES = 16
CHUNK = 4
N_WORKERS = 32
ROWS_PER_WORKER = SEQ // N_WORKERS
N_CHUNKS = ROWS_PER_WORKER // CHUNK


def _sc_body(emb_hbm, pe_hbm, out_hbm, e0, e1, p0, p1, si0, si1, *, scale):
    wid = lax.axis_index("s") * 2 + lax.axis_index("c")
    base = wid * ROWS_PER_WORKER
    ebufs, pbufs, sis = (e0, e1), (p0, p1), (si0, si1)

    def start_in(g, b):
        r0 = base + g * CHUNK
        pltpu.async_copy(emb_hbm.at[pl.ds(r0, CHUNK)], ebufs[b], sis[b])
        pltpu.async_copy(pe_hbm.at[pl.ds(r0, CHUNK)], pbufs[b], sis[b])

    def wait_in(b):
        pltpu.make_async_copy(emb_hbm.at[pl.ds(0, CHUNK)], ebufs[b], sis[b]).wait()
        pltpu.make_async_copy(pe_hbm.at[pl.ds(0, CHUNK)], pbufs[b], sis[b]).wait()

    def compute(b):
        ebuf, pbuf = ebufs[b], pbufs[b]

        def row_body(s):
            for bb in range(B):
                for k in range(DIM // LANES):
                    sl = pl.ds(k * LANES, LANES)
                    ebuf[s, bb, sl] = ebuf[s, bb, sl] * scale + pbuf[s, 0, sl]

        pl.loop(0, CHUNK)(row_body)

    start_in(0, 0)

    def chunk_body(g):
        b = lax.rem(g, 2)

        def even_path():
            wait_in(0)

            @pl.when(g < N_CHUNKS - 1)
            def _():
                start_in(g + 1, 1)

            compute(0)
            pltpu.sync_copy(ebufs[0], out_hbm.at[pl.ds(base + g * CHUNK, CHUNK)])

        def odd_path():
            wait_in(1)

            @pl.when(g < N_CHUNKS - 1)
            def _():
                start_in(g + 1, 0)

            compute(1)
            pltpu.sync_copy(ebufs[1], out_hbm.at[pl.ds(base + g * CHUNK, CHUNK)])

        lax.cond(b == 0, even_path, odd_path)

    pl.loop(0, N_CHUNKS)(chunk_body)


def kernel(emb, src_org, pe):
    del src_org
    seq, b, dim = emb.shape
    scale = math.sqrt(pe.shape[-1])

    mesh = plsc.VectorSubcoreMesh(core_axis_name="c", subcore_axis_name="s")
    sc_call = functools.partial(
        pl.kernel,
        mesh=mesh,
        out_type=jax.ShapeDtypeStruct((seq, b, dim), emb.dtype),
        scratch_types=[
            pltpu.VMEM((CHUNK, b, dim), jnp.float32),
            pltpu.VMEM((CHUNK, b, dim), jnp.float32),
            pltpu.VMEM((CHUNK, 1, dim), jnp.float32),
            pltpu.VMEM((CHUNK, 1, dim), jnp.float32),
            pltpu.SemaphoreType.DMA,
            pltpu.SemaphoreType.DMA,
        ],
    )(functools.partial(_sc_body, scale=scale))
    return sc_call(emb, pe)


# speedup vs baseline: 8.8257x; 8.8257x over previous
import math

import jax
import jax.numpy as jnp
from jax.experimental import pallas as pl


def _pe_add_block(emb_ref, pe_ref, out_ref, *, scale):
    out_ref[...] = emb_ref[...] * scale + pe_ref[...]


def kernel(emb, src_org, pe):
    del src_org
    seq, b, dim = emb.shape
    scale = math.sqrt(pe.shape[-1])

    block_s = 256
    grid = (seq // block_s,)

    return pl.pallas_call(
        lambda e, p, o: _pe_add_block(e, p, o, scale=scale),
        grid=grid,
        in_specs=[
            pl.BlockSpec((block_s, b, dim), lambda i: (i, 0, 0)),
            pl.BlockSpec((block_s, 1, dim), lambda i: (i, 0, 0)),
        ],
        out_specs=pl.BlockSpec((block_s, b, dim), lambda i: (i, 0, 0)),
        out_shape=jax.ShapeDtypeStruct((seq, b, dim), emb.dtype),
    )(emb, pe)
